# baseline (speedup 1.0000x reference)
import functools

import jax
import jax.numpy as jnp
from jax.experimental import pallas as pl

_N = 262144
_D = 64
_V = 100
_MC = 10
_BLK = 2048
_VP = 128
_MCP = 16


def _body(x_ref, w_ref, th_ref, out_ref):
    votes = jnp.sign(
        jax.lax.dot_general(
            x_ref[...], w_ref[...],
            (((1,), (0,)), ((), ())),
            preferred_element_type=jnp.float32,
        )
    )
    labels = jax.lax.dot_general(
        votes, th_ref[...],
        (((1,), (1,)), ((), ())),
        preferred_element_type=jnp.float32,
    )
    col = jax.lax.broadcasted_iota(jnp.int32, labels.shape, 1)
    ge = jnp.where((labels >= 0.0) & (col < _MC), 1.0, 0.0)
    cnt = jnp.sum(ge, axis=1, keepdims=True)
    out_ref[...] = jnp.concatenate(
        [(_MC - cnt) * (1.0 / _MC), cnt * (1.0 / _MC)], axis=1
    )


@jax.jit
def kernel(x, W, thetas):
    w_pad = jnp.zeros((_D, _VP), jnp.float32).at[:, :_V].set(W)
    th_pad = jnp.zeros((_MCP, _VP), jnp.float32).at[:_MC, :_V].set(thetas)
    return pl.pallas_call(
        _body,
        grid=(_N // _BLK,),
        in_specs=[
            pl.BlockSpec((_BLK, _D), lambda i: (i, 0)),
            pl.BlockSpec((_D, _VP), lambda i: (0, 0)),
            pl.BlockSpec((_MCP, _VP), lambda i: (0, 0)),
        ],
        out_specs=pl.BlockSpec((_BLK, 2), lambda i: (i, 0)),
        out_shape=jax.ShapeDtypeStruct((_N, 2), jnp.float32),
    )(x, w_pad, th_pad)


# speedup vs baseline: 1.2097x; 1.2097x over previous
import jax
import jax.numpy as jnp
import numpy as np
from jax.experimental import pallas as pl

_N = 262144
_D = 64
_V = 100
_MC = 10
_BLK = 8192
_VP = 128
_MCP = 16

_A_np = np.zeros((_MCP, 2), np.float32)
_A_np[:_MC, 0] = -1.0 / _MC
_A_np[_MC, 0] = 1.0
_A_np[:_MC, 1] = 1.0 / _MC


def _body(x_ref, w_ref, th_ref, a_ref, out_ref):
    acc = jax.lax.dot_general(
        x_ref[...], w_ref[...],
        (((1,), (0,)), ((), ())),
        preferred_element_type=jnp.float32,
    )
    votes = jnp.where(acc >= 0.0, 1.0, -1.0)
    labels = jax.lax.dot_general(
        votes, th_ref[...],
        (((1,), (1,)), ((), ())),
        preferred_element_type=jnp.float32,
    )
    ge = jnp.where(labels >= 0.0, 1.0, 0.0)
    out_ref[...] = jax.lax.dot_general(
        ge, a_ref[...],
        (((1,), (0,)), ((), ())),
        preferred_element_type=jnp.float32,
    )


@jax.jit
def kernel(x, W, thetas):
    w_pad = jnp.zeros((_D, _VP), jnp.float32).at[:, :_V].set(W)
    th_pad = jnp.zeros((_MCP, _VP), jnp.float32).at[:_MC, :_V].set(thetas)
    a = jnp.asarray(_A_np)
    return pl.pallas_call(
        _body,
        grid=(_N // _BLK,),
        in_specs=[
            pl.BlockSpec((_BLK, _D), lambda i: (i, 0)),
            pl.BlockSpec((_D, _VP), lambda i: (0, 0)),
            pl.BlockSpec((_MCP, _VP), lambda i: (0, 0)),
            pl.BlockSpec((_MCP, 2), lambda i: (0, 0)),
        ],
        out_specs=pl.BlockSpec((_BLK, 2), lambda i: (i, 0)),
        out_shape=jax.ShapeDtypeStruct((_N, 2), jnp.float32),
    )(x, w_pad, th_pad, a)


# speedup vs baseline: 1.9616x; 1.6215x over previous
import jax
import jax.numpy as jnp
import numpy as np
from jax.experimental import pallas as pl
from jax.experimental.pallas import tpu as pltpu

_N = 262144
_D = 64
_V = 100
_MC = 10
_BLK = 8192
_VP = 128
_MCP = 16

_A_np = np.zeros((_MCP, 2), np.float32)
_A_np[:_MC, 0] = -1.0 / _MC
_A_np[_MC, 0] = 1.0
_A_np[:_MC, 1] = 1.0 / _MC


def _body(x_ref, w_ref, th_ref, a_ref, out_ref):
    acc = jax.lax.dot_general(
        x_ref[...], w_ref[...],
        (((1,), (0,)), ((), ())),
        preferred_element_type=jnp.float32,
    )
    votes = jnp.where(acc >= 0.0, 1.0, -1.0)
    labels = jax.lax.dot_general(
        votes, th_ref[...],
        (((1,), (1,)), ((), ())),
        preferred_element_type=jnp.float32,
    )
    ge = jnp.where(labels >= 0.0, 1.0, 0.0)
    out_ref[...] = jax.lax.dot_general(
        a_ref[...], ge,
        (((0,), (1,)), ((), ())),
        preferred_element_type=jnp.float32,
    )


@jax.jit
def kernel(x, W, thetas):
    w_pad = jnp.zeros((_D, _VP), jnp.float32).at[:, :_V].set(W)
    th_pad = jnp.zeros((_MCP, _VP), jnp.float32).at[:_MC, :_V].set(thetas)
    a = jnp.asarray(_A_np)
    return pl.pallas_call(
        _body,
        grid=(_N // _BLK,),
        in_specs=[
            pl.BlockSpec((_BLK, _D), lambda i: (i, 0)),
            pl.BlockSpec((_D, _VP), lambda i: (0, 0)),
            pl.BlockSpec((_MCP, _VP), lambda i: (0, 0)),
            pl.BlockSpec((_MCP, 2), lambda i: (0, 0)),
        ],
        out_specs=pl.BlockSpec((2, _BLK), lambda i: (0, i)),
        out_shape=jax.ShapeDtypeStruct((2, _N), jnp.float32),
        compiler_params=pltpu.CompilerParams(
            dimension_semantics=(pltpu.PARALLEL,),
        ),
    )(x, w_pad, th_pad, a).T


# speedup vs baseline: 2.0629x; 1.0516x over previous
import jax
import jax.numpy as jnp
import numpy as np
from jax.experimental import pallas as pl
from jax.experimental.pallas import tpu as pltpu

_N = 262144
_D = 64
_V = 100
_MC = 10
_BLK = 16384
_VP = 128
_MCP = 16

_A_np = np.zeros((_MCP, 2), np.float32)
_A_np[:_MC, 0] = -1.0 / _MC
_A_np[_MC, 0] = 1.0
_A_np[:_MC, 1] = 1.0 / _MC


def _body(x_ref, w_ref, th_ref, a_ref, out_ref):
    acc = jax.lax.dot_general(
        x_ref[...], w_ref[...],
        (((1,), (0,)), ((), ())),
        preferred_element_type=jnp.float32,
    )
    acc_bits = jax.lax.bitcast_convert_type(acc, jnp.uint32)
    votes = jax.lax.bitcast_convert_type(
        (acc_bits & jnp.uint32(0x80000000)) | jnp.uint32(0x3F800000),
        jnp.float32,
    )
    labels = jax.lax.dot_general(
        votes, th_ref[...],
        (((1,), (1,)), ((), ())),
        preferred_element_type=jnp.float32,
    )
    ge = jnp.where(labels >= 0.0, 1.0, 0.0)
    out_ref[...] = jax.lax.dot_general(
        a_ref[...], ge,
        (((0,), (1,)), ((), ())),
        preferred_element_type=jnp.float32,
    )


@jax.jit
def kernel(x, W, thetas):
    w_pad = jnp.zeros((_D, _VP), jnp.float32).at[:, :_V].set(W)
    th_pad = jnp.zeros((_MCP, _VP), jnp.float32).at[:_MC, :_V].set(thetas)
    a = jnp.asarray(_A_np)
    return pl.pallas_call(
        _body,
        grid=(_N // _BLK,),
        in_specs=[
            pl.BlockSpec((_BLK, _D), lambda i: (i, 0)),
            pl.BlockSpec((_D, _VP), lambda i: (0, 0)),
            pl.BlockSpec((_MCP, _VP), lambda i: (0, 0)),
            pl.BlockSpec((_MCP, 2), lambda i: (0, 0)),
        ],
        out_specs=pl.BlockSpec((2, _BLK), lambda i: (0, i)),
        out_shape=jax.ShapeDtypeStruct((2, _N), jnp.float32),
        compiler_params=pltpu.CompilerParams(
            dimension_semantics=(pltpu.PARALLEL,),
        ),
    )(x, w_pad, th_pad, a).T
